# baseline (speedup 1.0000x reference)
import functools

import jax
import jax.numpy as jnp
from jax import lax
from jax.experimental import pallas as pl
from jax.experimental.pallas import tpu as pltpu
from jax.experimental.pallas import tpu_sc as plsc

_PREC = lax.Precision.HIGHEST
_S = 7
_H = 224
_CH = 32
_NW = 32


def _contract_body(wcan_ref, wss_ref, f_ref, g_ref):
    dn = (((0,), (0,)), ((), ()))
    g_ref[0, pl.ds(0, 1)] = lax.dot_general(
        wcan_ref[...], f_ref[0], dn, precision=_PREC,
        preferred_element_type=jnp.float32)
    g_ref[0, pl.ds(1, 2)] = lax.dot_general(
        wss_ref[...], f_ref[0], dn, precision=_PREC,
        preferred_element_type=jnp.float32)


def _round_half_even_i32(x):
    n0 = x.astype(jnp.int32)
    r = x - n0.astype(jnp.float32)
    odd = (n0 & 1) == 1
    up = (r > 0.5) | ((r == 0.5) & odd)
    return n0 + jnp.where(up, jnp.int32(1), jnp.int32(0))


def _sc_body(gmaps, par, tvec, d_out, can_out, ss_out,
             maps_v, tvec_v, par_v, iy_v, ix_v, dg8_v, selx_v, sely_v,
             rows_v, drows_v, ss_v, sem0):
    f32 = jnp.float32
    i32 = jnp.int32
    wid = lax.axis_index("s") * 2 + lax.axis_index("c")

    pltpu.sync_copy(gmaps, maps_v)
    pltpu.sync_copy(tvec, tvec_v)

    lane = lax.broadcasted_iota(i32, (16,), 0)
    zero16 = jnp.zeros((16,), f32)
    for sl in range(4):
        for cx in range(14):
            rows_v[sl, 7, pl.ds(cx * 16, 16)] = zero16

    def one_instance(j, _):
        v = wid + _NW * j

        @pl.when(v < 126)
        def _():
            pltpu.sync_copy(par.at[v], par_v)
            pv = par_v[...]
            b0 = pv[0]
            b1 = pv[1]
            b2 = pv[2]
            b3 = pv[3]
            validf = pv[4]
            b_idx = pv[5].astype(i32)
            b_can0 = pv[6]
            b_ss0 = pv[7]
            b_ss1 = pv[8]

            eps = jnp.float32(1e-3)
            x1 = jnp.minimum(b0, b2)
            x2 = jnp.maximum(b0, b2) + eps
            y1 = jnp.minimum(b1, b3)
            y2 = jnp.maximum(b1, b3) + eps

            tv = tvec_v[...]
            ys = y1 + (y2 - y1) * tv
            xs = x1 + (x2 - x1) * tv
            iy = jnp.clip(_round_half_even_i32(ys * jnp.float32(55.0)), 0, 55)
            ix = jnp.clip(_round_half_even_i32(xs * jnp.float32(55.0)), 0, 55)
            iy_v[...] = iy
            ix_v[...] = ix

            bvec = jnp.full((16,), b_idx, i32)
            f0 = jnp.zeros((16,), i32)
            acc0 = zero16
            acc1 = zero16
            for c in range(4):
                l = lane + (16 * c)
                p = jnp.right_shift(l, 3)
                q = jnp.bitwise_and(l, 7)
                okm = (p < _S) & (q < _S)
                iyp = plsc.load_gather(iy_v, [p])
                ixq = plsc.load_gather(ix_v, [q])
                vc = plsc.load_gather(maps_v, [bvec, f0, iyp, ixq])
                v0 = plsc.load_gather(maps_v, [bvec, f0 + 1, iyp, ixq])
                v1 = plsc.load_gather(maps_v, [bvec, f0 + 2, iyp, ixq])
                okf = jnp.where(okm, jnp.float32(1.0), jnp.float32(0.0))
                acc0 = acc0 + v0 * okf
                acc1 = acc1 + v1 * okf
                dg8_v[pl.ds(16 * c, 16)] = (vc + b_can0) * okf * validf

            inv49 = jnp.float32(1.0 / 49.0)
            scale = (jnp.sum(acc0) * inv49 + b_ss0) * validf
            shift = (jnp.sum(acc1) * inv49 + b_ss1) * validf

            inv223 = jnp.float32(223.0)
            dx = x2 - x1
            dy = y2 - y1
            six = jnp.float32(6.0)
            for c in range(14):
                xq = (lane + (16 * c)).astype(f32) / inv223
                ux = (xq - x1) / dx
                mx = (ux >= 0) & (ux <= 1)
                gx = jnp.clip(_round_half_even_i32(ux * six), 0, 6)
                selx_v[pl.ds(16 * c, 16)] = jnp.where(mx, gx, 7)
                uy = (xq - y1) / dy
                my = (uy >= 0) & (uy <= 1)
                gy = jnp.clip(_round_half_even_i32(uy * six), 0, 6)
                sely_v[pl.ds(16 * c, 16)] = jnp.where(my, gy, 7)

            dzero = jnp.maximum(shift, jnp.float32(0.001))
            for k in range(_S):
                krow8 = jnp.full((16,), k * 8, i32)
                for cx in range(14):
                    s = pl.ds(cx * 16, 16)
                    gxv = selx_v[s]
                    rv = plsc.load_gather(dg8_v, [krow8 + gxv])
                    rows_v[j, k, s] = rv
                    drows_v[j, k, s] = jnp.maximum(
                        rv * scale + shift, jnp.float32(0.001))
            dz16 = jnp.full((16,), dzero, f32)
            for cx in range(14):
                drows_v[j, 7, pl.ds(cx * 16, 16)] = dz16

            def issue_rows(grp, carry):
                selv = sely_v[pl.ds(grp * 16, 16)]
                ybase = grp * 16
                for i in range(16):
                    sel = selv[i]
                    y = ybase + i
                    pltpu.async_copy(
                        drows_v.at[j, sel], d_out.at[v, y], sem0)
                    pltpu.async_copy(
                        rows_v.at[j, sel], can_out.at[v, y], sem0)
                return carry

            lax.fori_loop(0, _H // 16, issue_rows, 0)

            ssv = jnp.where(lane == 0, scale,
                            jnp.where(lane == 1, shift, jnp.float32(0.0)))
            ss_v[...] = ssv
            pltpu.sync_copy(ss_v, ss_out.at[v])

        @pl.when((j > 0) & (v - _NW < 126))
        def _drain_prev():
            def drain(_, carry):
                pltpu.make_async_copy(
                    d_out.at[0, pl.ds(0, 16)],
                    d_out.at[0, pl.ds(0, 16)], sem0).wait()
                return carry
            lax.fori_loop(0, 2 * (_H // 16), drain, 0)

        return 0

    lax.fori_loop(0, 4, one_instance, 0)

    @pl.when(wid < 30)
    def _drain_last():
        def drain(_, carry):
            pltpu.make_async_copy(
                d_out.at[0, pl.ds(0, 16)],
                d_out.at[0, pl.ds(0, 16)], sem0).wait()
            return carry
        lax.fori_loop(0, 2 * (_H // 16), drain, 0)


def kernel(depth, context, input_feature_map, bin_num, min_depth, max_depth,
           masks, instances, boxes, labels, W_ss, b_ss, W_can, b_can):
    f32 = jnp.float32
    B, I, h, w = instances.shape
    _, C, Hf, Wf = input_feature_map.shape
    N = B * I

    gmaps = pl.pallas_call(
        _contract_body,
        grid=(B,),
        in_specs=[
            pl.BlockSpec((C, 1), lambda b: (0, 0)),
            pl.BlockSpec((C, 2), lambda b: (0, 0)),
            pl.BlockSpec((1, C, Hf, Wf), lambda b: (b, 0, 0, 0)),
        ],
        out_specs=pl.BlockSpec((1, 3, Hf, Wf), lambda b: (b, 0, 0, 0)),
        out_shape=jax.ShapeDtypeStruct((B, 3, Hf, Wf), f32),
    )(W_can, W_ss, input_feature_map)

    valid = (labels.reshape(N, 1) != 0).astype(f32)
    batchf = jnp.repeat(jnp.arange(B, dtype=f32), I).reshape(N, 1)
    par = jnp.concatenate(
        [boxes.reshape(N, 4), valid, batchf,
         jnp.broadcast_to(b_can.reshape(1, 1), (N, 1)),
         jnp.broadcast_to(b_ss.reshape(1, 2), (N, 2)),
         jnp.zeros((N, 7), f32)], axis=1)
    par = jnp.concatenate([par, jnp.zeros((2, 16), f32)], axis=0)
    tvec = jnp.concatenate(
        [jnp.linspace(0.0, 1.0, _S).astype(f32), jnp.zeros((9,), f32)])

    mesh = plsc.VectorSubcoreMesh(core_axis_name="c", subcore_axis_name="s")
    sc = functools.partial(
        pl.kernel, mesh=mesh,
        compiler_params=pltpu.CompilerParams(needs_layout_passes=False),
        out_type=[
            jax.ShapeDtypeStruct((N, h, w), f32),
            jax.ShapeDtypeStruct((N, h, w), f32),
            jax.ShapeDtypeStruct((N + 2, 16), f32),
        ],
        scratch_types=[
            pltpu.VMEM((B, 3, Hf, Wf), f32),
            pltpu.VMEM((16,), f32),
            pltpu.VMEM((16,), f32),
            pltpu.VMEM((16,), jnp.int32),
            pltpu.VMEM((16,), jnp.int32),
            pltpu.VMEM((64,), f32),
            pltpu.VMEM((_H,), jnp.int32),
            pltpu.VMEM((_H,), jnp.int32),
            pltpu.VMEM((4, 8, _H), f32),
            pltpu.VMEM((4, 8, _H), f32),
            pltpu.VMEM((16,), f32),
            pltpu.SemaphoreType.DMA,
        ],
    )(_sc_body)
    d, can, ss = sc(gmaps, par, tvec)

    d = d.reshape(B, I, h, w)
    can = can.reshape(B, I, h, w)
    scale = ss[:N, 0].reshape(B, I)
    shift = ss[:N, 1].reshape(B, I)
    return (d, can, scale, shift)


# speedup vs baseline: 1.1809x; 1.1809x over previous
import functools

import jax
import jax.numpy as jnp
from jax import lax
from jax.experimental import pallas as pl
from jax.experimental.pallas import tpu as pltpu
from jax.experimental.pallas import tpu_sc as plsc

_PREC = lax.Precision.HIGHEST
_S = 7
_H = 224
_CH = 32
_NW = 32


def _contract_body(w_ref, f_ref, g_ref):
    g_ref[0] = jnp.dot(w_ref[...], f_ref[0], precision=_PREC,
                       preferred_element_type=jnp.float32)


def _round_half_even_i32(x):
    n0 = x.astype(jnp.int32)
    r = x - n0.astype(jnp.float32)
    odd = (n0 & 1) == 1
    up = (r > 0.5) | ((r == 0.5) & odd)
    return n0 + jnp.where(up, jnp.int32(1), jnp.int32(0))


def _sc_body(gmaps, par, tvec, d_out, can_out, ss_out,
             maps_v, tvec_v, par_v, iy_v, ix_v, dg8_v, selx_v, sely_v,
             rows_v, drows_v, ss_v, sem0):
    f32 = jnp.float32
    i32 = jnp.int32
    wid = lax.axis_index("s") * 2 + lax.axis_index("c")

    pltpu.sync_copy(gmaps.at[0, pl.ds(0, 3)], maps_v.at[pl.ds(0, 3)])
    pltpu.sync_copy(gmaps.at[1, pl.ds(0, 3)], maps_v.at[pl.ds(3, 3)])
    pltpu.sync_copy(tvec, tvec_v)

    lane = lax.broadcasted_iota(i32, (16,), 0)
    zero16 = jnp.zeros((16,), f32)
    for sl in range(4):
        for cx in range(14):
            rows_v[sl, 7, pl.ds(cx * 16, 16)] = zero16

    def one_instance(j, _):
        v = wid + _NW * j

        @pl.when(v < 126)
        def _():
            pltpu.sync_copy(par.at[v], par_v)
            pv = par_v[...]
            b0 = pv[0]
            b1 = pv[1]
            b2 = pv[2]
            b3 = pv[3]
            validf = pv[4]
            b_idx = pv[5].astype(i32)
            b_can0 = pv[6]
            b_ss0 = pv[7]
            b_ss1 = pv[8]

            eps = jnp.float32(1e-3)
            x1 = jnp.minimum(b0, b2)
            x2 = jnp.maximum(b0, b2) + eps
            y1 = jnp.minimum(b1, b3)
            y2 = jnp.maximum(b1, b3) + eps

            tv = tvec_v[...]
            ys = y1 + (y2 - y1) * tv
            xs = x1 + (x2 - x1) * tv
            iy = jnp.clip(_round_half_even_i32(ys * jnp.float32(55.0)), 0, 55)
            ix = jnp.clip(_round_half_even_i32(xs * jnp.float32(55.0)), 0, 55)
            iy_v[...] = iy
            ix_v[...] = ix

            row_can = jnp.full((16,), b_idx * 3, i32)
            acc0 = zero16
            acc1 = zero16
            for c in range(4):
                l = lane + (16 * c)
                p = jnp.right_shift(l, 3)
                q = jnp.bitwise_and(l, 7)
                okm = (p < _S) & (q < _S)
                iyp = plsc.load_gather(iy_v, [p])
                ixq = plsc.load_gather(ix_v, [q])
                col = iyp * 56 + ixq
                vc = plsc.load_gather(maps_v, [row_can, col])
                v0 = plsc.load_gather(maps_v, [row_can + 1, col])
                v1 = plsc.load_gather(maps_v, [row_can + 2, col])
                okf = jnp.where(okm, jnp.float32(1.0), jnp.float32(0.0))
                acc0 = acc0 + v0 * okf
                acc1 = acc1 + v1 * okf
                dg8_v[pl.ds(16 * c, 16)] = (vc + b_can0) * okf * validf

            inv49 = jnp.float32(1.0 / 49.0)
            scale = (jnp.sum(acc0) * inv49 + b_ss0) * validf
            shift = (jnp.sum(acc1) * inv49 + b_ss1) * validf

            inv223 = jnp.float32(223.0)
            dx = x2 - x1
            dy = y2 - y1
            six = jnp.float32(6.0)
            for c in range(14):
                xq = (lane + (16 * c)).astype(f32) / inv223
                ux = (xq - x1) / dx
                mx = (ux >= 0) & (ux <= 1)
                gx = jnp.clip(_round_half_even_i32(ux * six), 0, 6)
                selx_v[pl.ds(16 * c, 16)] = jnp.where(mx, gx, 7)
                uy = (xq - y1) / dy
                my = (uy >= 0) & (uy <= 1)
                gy = jnp.clip(_round_half_even_i32(uy * six), 0, 6)
                sely_v[pl.ds(16 * c, 16)] = jnp.where(my, gy, 7)

            dzero = jnp.maximum(shift, jnp.float32(0.001))
            for k in range(_S):
                krow8 = jnp.full((16,), k * 8, i32)
                for cx in range(14):
                    s = pl.ds(cx * 16, 16)
                    gxv = selx_v[s]
                    rv = plsc.load_gather(dg8_v, [krow8 + gxv])
                    rows_v[j, k, s] = rv
                    drows_v[j, k, s] = jnp.maximum(
                        rv * scale + shift, jnp.float32(0.001))
            dz16 = jnp.full((16,), dzero, f32)
            for cx in range(14):
                drows_v[j, 7, pl.ds(cx * 16, 16)] = dz16

            def issue_rows(grp, carry):
                selv = sely_v[pl.ds(grp * 16, 16)]
                ybase = grp * 16
                for i in range(16):
                    sel = selv[i]
                    y = ybase + i
                    pltpu.async_copy(
                        drows_v.at[j, sel], d_out.at[v, y], sem0)
                    pltpu.async_copy(
                        rows_v.at[j, sel], can_out.at[v, y], sem0)
                return carry

            lax.fori_loop(0, _H // 16, issue_rows, 0)

            ssv = jnp.where(lane == 0, scale,
                            jnp.where(lane == 1, shift, jnp.float32(0.0)))
            ss_v[...] = ssv
            pltpu.sync_copy(ss_v, ss_out.at[v])

        @pl.when((j > 0) & (v - _NW < 126))
        def _drain_prev():
            def drain(_, carry):
                pltpu.make_async_copy(
                    d_out.at[0, pl.ds(0, 16)],
                    d_out.at[0, pl.ds(0, 16)], sem0).wait()
                return carry
            lax.fori_loop(0, 2 * (_H // 16), drain, 0)

        return 0

    lax.fori_loop(0, 4, one_instance, 0)

    @pl.when(wid < 30)
    def _drain_last():
        def drain(_, carry):
            pltpu.make_async_copy(
                d_out.at[0, pl.ds(0, 16)],
                d_out.at[0, pl.ds(0, 16)], sem0).wait()
            return carry
        lax.fori_loop(0, 2 * (_H // 16), drain, 0)


def kernel(depth, context, input_feature_map, bin_num, min_depth, max_depth,
           masks, instances, boxes, labels, W_ss, b_ss, W_can, b_can):
    f32 = jnp.float32
    B, I, h, w = instances.shape
    _, C, Hf, Wf = input_feature_map.shape
    N = B * I

    Wcat = jnp.concatenate(
        [W_can[:, 0:1], W_ss[:, 0:1], W_ss[:, 1:2],
         jnp.zeros((C, 5), f32)], axis=1).T
    fmap2 = input_feature_map.reshape(B, C, Hf * Wf)
    g = pl.pallas_call(
        _contract_body,
        grid=(B,),
        in_specs=[
            pl.BlockSpec((8, C), lambda b: (0, 0)),
            pl.BlockSpec((1, C, Hf * Wf), lambda b: (b, 0, 0)),
        ],
        out_specs=pl.BlockSpec((1, 8, Hf * Wf), lambda b: (b, 0, 0)),
        out_shape=jax.ShapeDtypeStruct((B, 8, Hf * Wf), f32),
    )(Wcat, fmap2)

    valid = (labels.reshape(N, 1) != 0).astype(f32)
    batchf = jnp.repeat(jnp.arange(B, dtype=f32), I).reshape(N, 1)
    par = jnp.concatenate(
        [boxes.reshape(N, 4), valid, batchf,
         jnp.broadcast_to(b_can.reshape(1, 1), (N, 1)),
         jnp.broadcast_to(b_ss.reshape(1, 2), (N, 2)),
         jnp.zeros((N, 7), f32)], axis=1)
    par = jnp.concatenate([par, jnp.zeros((2, 16), f32)], axis=0)
    tvec = jnp.concatenate(
        [jnp.linspace(0.0, 1.0, _S).astype(f32), jnp.zeros((9,), f32)])

    mesh = plsc.VectorSubcoreMesh(core_axis_name="c", subcore_axis_name="s")
    sc = functools.partial(
        pl.kernel, mesh=mesh,
        compiler_params=pltpu.CompilerParams(needs_layout_passes=False),
        out_type=[
            jax.ShapeDtypeStruct((N, h, w), f32),
            jax.ShapeDtypeStruct((N, h, w), f32),
            jax.ShapeDtypeStruct((N + 2, 16), f32),
        ],
        scratch_types=[
            pltpu.VMEM((2 * 3, Hf * Wf), f32),
            pltpu.VMEM((16,), f32),
            pltpu.VMEM((16,), f32),
            pltpu.VMEM((16,), jnp.int32),
            pltpu.VMEM((16,), jnp.int32),
            pltpu.VMEM((64,), f32),
            pltpu.VMEM((_H,), jnp.int32),
            pltpu.VMEM((_H,), jnp.int32),
            pltpu.VMEM((4, 8, _H), f32),
            pltpu.VMEM((4, 8, _H), f32),
            pltpu.VMEM((16,), f32),
            pltpu.SemaphoreType.DMA,
        ],
    )(_sc_body)
    d, can, ss = sc(g, par, tvec)

    d = d.reshape(B, I, h, w)
    can = can.reshape(B, I, h, w)
    scale = ss[:N, 0].reshape(B, I)
    shift = ss[:N, 1].reshape(B, I)
    return (d, can, scale, shift)


# speedup vs baseline: 1.2072x; 1.0223x over previous
import functools

import jax
import jax.numpy as jnp
from jax import lax
from jax.experimental import pallas as pl
from jax.experimental.pallas import tpu as pltpu
from jax.experimental.pallas import tpu_sc as plsc

_PREC = lax.Precision.HIGHEST
_S = 7
_H = 224
_CH = 32
_NW = 32


def _contract_body(w_ref, f_ref, g_ref):
    g_ref[0] = jnp.dot(w_ref[...], f_ref[0], precision=_PREC,
                       preferred_element_type=jnp.float32)


def _round_half_even_i32(x):
    n0 = x.astype(jnp.int32)
    r = x - n0.astype(jnp.float32)
    odd = (n0 & 1) == 1
    up = (r > 0.5) | ((r == 0.5) & odd)
    return n0 + jnp.where(up, jnp.int32(1), jnp.int32(0))


def _sc_body(gmaps, par, tvec, d_out, can_out, ss_out,
             maps_v, tvec_v, par_v, iy_v, ix_v, dg8_v, selx_v, sely_v,
             rows_v, drows_v, ss_v, sem0):
    f32 = jnp.float32
    i32 = jnp.int32
    wid = lax.axis_index("s") * 2 + lax.axis_index("c")

    pltpu.sync_copy(gmaps.at[0, pl.ds(0, 3)], maps_v.at[pl.ds(0, 3)])
    pltpu.sync_copy(gmaps.at[1, pl.ds(0, 3)], maps_v.at[pl.ds(3, 3)])
    pltpu.sync_copy(tvec, tvec_v)
    pltpu.sync_copy(par, par_v)
    tv_all = tvec_v[...]
    b_can0 = tv_all[7]
    b_ss0 = tv_all[8]
    b_ss1 = tv_all[9]

    lane = lax.broadcasted_iota(i32, (16,), 0)
    zero16 = jnp.zeros((16,), f32)
    for sl in range(4):
        for cx in range(14):
            rows_v[sl, 7, pl.ds(cx * 16, 16)] = zero16

    def one_instance(j, _):
        v = wid + _NW * j

        @pl.when(v < 126)
        def _():
            pv = par_v[pl.ds(8 * v, 16)]
            b0 = pv[0]
            b1 = pv[1]
            b2 = pv[2]
            b3 = pv[3]
            validf = jnp.where(pv[4] != 0, jnp.float32(1.0), jnp.float32(0.0))
            b_idx = jnp.where(v >= 63, jnp.int32(1), jnp.int32(0))

            eps = jnp.float32(1e-3)
            x1 = jnp.minimum(b0, b2)
            x2 = jnp.maximum(b0, b2) + eps
            y1 = jnp.minimum(b1, b3)
            y2 = jnp.maximum(b1, b3) + eps

            tv = tvec_v[...]
            ys = y1 + (y2 - y1) * tv
            xs = x1 + (x2 - x1) * tv
            iy = jnp.clip(_round_half_even_i32(ys * jnp.float32(55.0)), 0, 55)
            ix = jnp.clip(_round_half_even_i32(xs * jnp.float32(55.0)), 0, 55)
            iy_v[...] = iy
            ix_v[...] = ix

            row_can = jnp.full((16,), b_idx * 3, i32)
            acc0 = zero16
            acc1 = zero16
            for c in range(4):
                l = lane + (16 * c)
                p = jnp.right_shift(l, 3)
                q = jnp.bitwise_and(l, 7)
                okm = (p < _S) & (q < _S)
                iyp = plsc.load_gather(iy_v, [p])
                ixq = plsc.load_gather(ix_v, [q])
                col = iyp * 56 + ixq
                vc = plsc.load_gather(maps_v, [row_can, col])
                v0 = plsc.load_gather(maps_v, [row_can + 1, col])
                v1 = plsc.load_gather(maps_v, [row_can + 2, col])
                okf = jnp.where(okm, jnp.float32(1.0), jnp.float32(0.0))
                acc0 = acc0 + v0 * okf
                acc1 = acc1 + v1 * okf
                dg8_v[pl.ds(16 * c, 16)] = (vc + b_can0) * okf * validf

            inv49 = jnp.float32(1.0 / 49.0)
            scale = (jnp.sum(acc0) * inv49 + b_ss0) * validf
            shift = (jnp.sum(acc1) * inv49 + b_ss1) * validf

            inv223 = jnp.float32(223.0)
            dx = x2 - x1
            dy = y2 - y1
            six = jnp.float32(6.0)
            for c in range(14):
                xq = (lane + (16 * c)).astype(f32) / inv223
                ux = (xq - x1) / dx
                mx = (ux >= 0) & (ux <= 1)
                gx = jnp.clip(_round_half_even_i32(ux * six), 0, 6)
                selx_v[pl.ds(16 * c, 16)] = jnp.where(mx, gx, 7)
                uy = (xq - y1) / dy
                my = (uy >= 0) & (uy <= 1)
                gy = jnp.clip(_round_half_even_i32(uy * six), 0, 6)
                sely_v[pl.ds(16 * c, 16)] = jnp.where(my, gy, 7)

            dzero = jnp.maximum(shift, jnp.float32(0.001))
            for k in range(_S):
                krow8 = jnp.full((16,), k * 8, i32)
                for cx in range(14):
                    s = pl.ds(cx * 16, 16)
                    gxv = selx_v[s]
                    rv = plsc.load_gather(dg8_v, [krow8 + gxv])
                    rows_v[j, k, s] = rv
                    drows_v[j, k, s] = jnp.maximum(
                        rv * scale + shift, jnp.float32(0.001))
            dz16 = jnp.full((16,), dzero, f32)
            for cx in range(14):
                drows_v[j, 7, pl.ds(cx * 16, 16)] = dz16

            def issue_rows(grp, carry):
                selv = sely_v[pl.ds(grp * 16, 16)]
                ybase = grp * 16
                for i in range(16):
                    sel = selv[i]
                    y = ybase + i
                    pltpu.async_copy(
                        drows_v.at[j, sel], d_out.at[v, y], sem0)
                    pltpu.async_copy(
                        rows_v.at[j, sel], can_out.at[v, y], sem0)
                return carry

            lax.fori_loop(0, _H // 16, issue_rows, 0)

            ssv = jnp.where(lane == 0, scale,
                            jnp.where(lane == 1, shift, jnp.float32(0.0)))
            ss_v[...] = ssv
            pltpu.sync_copy(ss_v, ss_out.at[v])

        @pl.when((j > 0) & (v - _NW < 126))
        def _drain_prev():
            def drain(_, carry):
                pltpu.make_async_copy(
                    d_out.at[0, pl.ds(0, 16)],
                    d_out.at[0, pl.ds(0, 16)], sem0).wait()
                return carry
            lax.fori_loop(0, 2 * (_H // 16), drain, 0)

        return 0

    lax.fori_loop(0, 4, one_instance, 0)

    @pl.when(wid < 30)
    def _drain_last():
        def drain(_, carry):
            pltpu.make_async_copy(
                d_out.at[0, pl.ds(0, 16)],
                d_out.at[0, pl.ds(0, 16)], sem0).wait()
            return carry
        lax.fori_loop(0, 2 * (_H // 16), drain, 0)


def kernel(depth, context, input_feature_map, bin_num, min_depth, max_depth,
           masks, instances, boxes, labels, W_ss, b_ss, W_can, b_can):
    f32 = jnp.float32
    B, I, h, w = instances.shape
    _, C, Hf, Wf = input_feature_map.shape
    N = B * I

    Wcat = jnp.concatenate(
        [W_can[:, 0:1], W_ss[:, 0:1], W_ss[:, 1:2],
         jnp.zeros((C, 5), f32)], axis=1).T
    fmap2 = input_feature_map.reshape(B, C, Hf * Wf)
    g = pl.pallas_call(
        _contract_body,
        grid=(B,),
        in_specs=[
            pl.BlockSpec((8, C), lambda b: (0, 0)),
            pl.BlockSpec((1, C, Hf * Wf), lambda b: (b, 0, 0)),
        ],
        out_specs=pl.BlockSpec((1, 8, Hf * Wf), lambda b: (b, 0, 0)),
        out_shape=jax.ShapeDtypeStruct((B, 8, Hf * Wf), f32),
    )(Wcat, fmap2)

    par = jnp.pad(
        jnp.concatenate(
            [boxes.reshape(N, 4), labels.reshape(N, 1).astype(f32)], axis=1),
        ((0, 2), (0, 3))).reshape((N + 2) * 8)
    tvec = jnp.concatenate(
        [jnp.linspace(0.0, 1.0, _S).astype(f32), b_can.reshape(-1),
         b_ss.reshape(-1), jnp.zeros((6,), f32)])

    mesh = plsc.VectorSubcoreMesh(core_axis_name="c", subcore_axis_name="s")
    sc = functools.partial(
        pl.kernel, mesh=mesh,
        compiler_params=pltpu.CompilerParams(needs_layout_passes=False),
        out_type=[
            jax.ShapeDtypeStruct((N, h, w), f32),
            jax.ShapeDtypeStruct((N, h, w), f32),
            jax.ShapeDtypeStruct((N + 2, 16), f32),
        ],
        scratch_types=[
            pltpu.VMEM((2 * 3, Hf * Wf), f32),
            pltpu.VMEM((16,), f32),
            pltpu.VMEM((1024,), f32),
            pltpu.VMEM((16,), jnp.int32),
            pltpu.VMEM((16,), jnp.int32),
            pltpu.VMEM((64,), f32),
            pltpu.VMEM((_H,), jnp.int32),
            pltpu.VMEM((_H,), jnp.int32),
            pltpu.VMEM((4, 8, _H), f32),
            pltpu.VMEM((4, 8, _H), f32),
            pltpu.VMEM((16,), f32),
            pltpu.SemaphoreType.DMA,
        ],
    )(_sc_body)
    d, can, ss = sc(g, par, tvec)

    d = d.reshape(B, I, h, w)
    can = can.reshape(B, I, h, w)
    scale = ss[:N, 0].reshape(B, I)
    shift = ss[:N, 1].reshape(B, I)
    return (d, can, scale, shift)
